# baseline (speedup 1.0000x reference)
import functools

import jax
import jax.numpy as jnp
from jax import lax
from jax.experimental import pallas as pl
from jax.experimental.pallas import tpu as pltpu
from jax.experimental.pallas import tpu_sc as plsc

N = 128
L = 16
RPW = 8
IHALF = N // 2
IB = 8

_mesh = plsc.VectorSubcoreMesh(core_axis_name="c", subcore_axis_name="s")
_params = pltpu.CompilerParams(needs_layout_passes=False)


@functools.partial(
    pl.kernel,
    out_type=jax.ShapeDtypeStruct((2 * 16 * L,), jnp.float32),
    mesh=_mesh,
    compiler_params=_params,
    scratch_types=[
        pltpu.VMEM((RPW, N), jnp.float32),
        pltpu.VMEM((2 * RPW * N,), jnp.float32),
        pltpu.VMEM((8, RPW, L), jnp.float32),
        pltpu.VMEM_SHARED((8, N, L), jnp.float32),
        pltpu.VMEM_SHARED((16 * L,), jnp.float32),
        pltpu.VMEM((N, L), jnp.float32),
        pltpu.VMEM((IHALF, N), jnp.float32),
        pltpu.VMEM((16 * L,), jnp.float32),
        pltpu.VMEM((L,), jnp.float32),
    ],
)
def _hausdorff_kernel(preds_hbm, targets_hbm, out_hbm, m_v, dbuf_v, g2row_v,
                      g2_sh, u_sh, g2c_v, ma_v, u_v, stage_v):
    d = lax.axis_index("c")
    s = lax.axis_index("s")

    @pl.when(d == 0)
    def _():
        pltpu.sync_copy(targets_hbm.at[pl.ds(s * RPW, RPW), :], m_v)

    @pl.when(d == 1)
    def _():
        pltpu.sync_copy(preds_hbm.at[pl.ds(s * RPW, RPW), :], m_v)

    lane = lax.iota(jnp.int32, L)
    row_l = lane & 7
    fwd = lane < 8

    def sweep(j, dist):
        col = jnp.where(fwd, j, (N - 1) - j)
        m = plsc.load_gather(m_v, [row_l, col])
        dist = jnp.where(m > 0.5, jnp.float32(0.0), dist + 1.0)
        plsc.store_scatter(dbuf_v, [lane * N + col], dist)
        return dist

    with jax.named_scope("p1_sweep"):
        lax.fori_loop(0, N, sweep, jnp.full((L,), jnp.inf, jnp.float32))

    with jax.named_scope("p1_square"):
        for r in range(RPW):
            for c in range(N // L):
                off = r * N + c * L
                mn = jnp.minimum(dbuf_v[pl.ds(off, L)],
                                 dbuf_v[pl.ds(RPW * N + off, L)])
                g2row_v[c, r, :] = mn * mn

    with jax.named_scope("p1_publish"):
        for c in range(N // L):
            pltpu.sync_copy(g2row_v.at[c], g2_sh.at[c].at[pl.ds(s * RPW, RPW), :])
    with jax.named_scope("bar1"):
        plsc.subcore_barrier()

    q = s % 8
    jc = q * L
    i0 = (s // 8) * IHALF
    with jax.named_scope("pull"):
        pltpu.sync_copy(g2_sh.at[q], g2c_v)

    with jax.named_scope("ma_dma"):
        @pl.when(d == 0)
        def _():
            pltpu.sync_copy(preds_hbm.at[pl.ds(i0, IHALF), :], ma_v)

        @pl.when(d == 1)
        def _():
            pltpu.sync_copy(targets_hbm.at[pl.ds(i0, IHALF), :], ma_v)

    with jax.named_scope("bound"):
        ub = jnp.full((L,), -jnp.inf, jnp.float32)
        for row in range(IHALF):
            m = ma_v[row, pl.ds(jc, L)]
            g = g2c_v[i0 + row, :]
            ub = jnp.maximum(ub, jnp.where(m > 0.5, g, -jnp.inf))

    with jax.named_scope("ured"):
        stage_v[...] = ub
        pltpu.sync_copy(stage_v, u_sh.at[pl.ds(s * L, L)])
        plsc.subcore_barrier()
        pltpu.sync_copy(u_sh, u_v)

        bv = u_v[pl.ds(0, L)]
        for c in range(1, 16):
            bv = jnp.maximum(bv, u_v[pl.ds(c * L, L)])
        bound = jnp.max(bv)

        radius = jnp.int32(0)
        for c in range(N // L):
            dv = (lane + (c * L + 1)).astype(jnp.float32)
            radius = radius + jnp.sum(jnp.where(dv * dv <= bound, 1, 0))

    inf16 = jnp.full((L,), jnp.inf, jnp.float32)
    best = jnp.full((L,), -jnp.inf, jnp.float32)
    _mp_scope = jax.named_scope("minplus")
    _mp_scope.__enter__()
    for ib in range(IHALF // IB):
        base_i = i0 + ib * IB
        lo = jnp.maximum(base_i - radius, 0)
        hi = jnp.minimum(base_i + (IB - 1) + radius + 1, N)

        def minplus(i2, accs):
            g = g2c_v[i2, :]
            out = []
            for k in range(IB):
                di = (i0 + ib * IB + k) - i2
                out.append(jnp.minimum(accs[k], g + (di * di).astype(jnp.float32)))
            return tuple(out)

        accs = lax.fori_loop(lo, hi, minplus, (inf16,) * IB)

        for k in range(IB):
            m = ma_v[ib * IB + k, pl.ds(jc, L)]
            best = jnp.maximum(best, jnp.where(m > 0.5, accs[k], -jnp.inf))

    _mp_scope.__exit__(None, None, None)
    with jax.named_scope("outw"):
        stage_v[...] = best
        pltpu.sync_copy(stage_v, out_hbm.at[pl.ds((d * 16 + s) * L, L)])


def kernel(preds, targets):
    partials = _hausdorff_kernel(preds, targets)
    max_min = jnp.max(partials.reshape(2, 16 * L), axis=1)
    hd = jnp.sqrt(max_min)
    return jnp.maximum(hd[0], hd[1])


# speedup vs baseline: 1.0621x; 1.0621x over previous
import functools

import jax
import jax.numpy as jnp
from jax import lax
from jax.experimental import pallas as pl
from jax.experimental.pallas import tpu as pltpu
from jax.experimental.pallas import tpu_sc as plsc

N = 128
L = 16
RPW = 8
IHALF = N // 2
IB = 8
WOFF = (2, 4, 6)

_mesh = plsc.VectorSubcoreMesh(core_axis_name="c", subcore_axis_name="s")
_params = pltpu.CompilerParams(needs_layout_passes=False)


@functools.partial(
    pl.kernel,
    out_type=jax.ShapeDtypeStruct((2 * 16 * L,), jnp.float32),
    mesh=_mesh,
    compiler_params=_params,
    scratch_types=[
        pltpu.VMEM((RPW, N), jnp.float32),
        pltpu.VMEM((2 * RPW * N,), jnp.float32),
        pltpu.VMEM((8, RPW, L), jnp.float32),
        pltpu.VMEM_SHARED((8, N, L), jnp.float32),
        pltpu.VMEM_SHARED((16 * L,), jnp.float32),
        pltpu.VMEM((N, L), jnp.float32),
        pltpu.VMEM((IHALF, N), jnp.float32),
        pltpu.VMEM((16 * L,), jnp.float32),
        pltpu.VMEM((L,), jnp.float32),
        pltpu.SemaphoreType.DMA,
        pltpu.SemaphoreType.DMA,
    ],
)
def _hausdorff_kernel(preds_hbm, targets_hbm, out_hbm, m_v, dbuf_v, g2row_v,
                      g2_sh, u_sh, g2c_v, ma_v, u_v, stage_v, sem_ma, sem_pub):
    d = lax.axis_index("c")
    s = lax.axis_index("s")
    i0_pre = (s // 8) * IHALF

    @pl.when(d == 0)
    def _():
        pltpu.async_copy(preds_hbm.at[pl.ds(i0_pre, IHALF), :], ma_v, sem_ma)

    @pl.when(d == 1)
    def _():
        pltpu.async_copy(targets_hbm.at[pl.ds(i0_pre, IHALF), :], ma_v, sem_ma)

    @pl.when(d == 0)
    def _():
        pltpu.sync_copy(targets_hbm.at[pl.ds(s * RPW, RPW), :], m_v)

    @pl.when(d == 1)
    def _():
        pltpu.sync_copy(preds_hbm.at[pl.ds(s * RPW, RPW), :], m_v)

    lane = lax.iota(jnp.int32, L)
    row_l = lane & 7
    fwd = lane < 8

    def sweep(j, dist):
        col = jnp.where(fwd, j, (N - 1) - j)
        m = plsc.load_gather(m_v, [row_l, col])
        dist = jnp.where(m > 0.5, jnp.float32(0.0), dist + 1.0)
        plsc.store_scatter(dbuf_v, [lane * N + col], dist)
        return dist

    lax.fori_loop(0, N, sweep, jnp.full((L,), jnp.inf, jnp.float32))

    for r in range(RPW):
        for c in range(N // L):
            off = r * N + c * L
            mn = jnp.minimum(dbuf_v[pl.ds(off, L)],
                             dbuf_v[pl.ds(RPW * N + off, L)])
            g2row_v[c, r, :] = mn * mn

    pubs = [pltpu.async_copy(g2row_v.at[c],
                             g2_sh.at[c].at[pl.ds(s * RPW, RPW), :], sem_pub)
            for c in range(N // L)]
    for p in pubs:
        p.wait()
    plsc.subcore_barrier()

    q = s % 8
    jc = q * L
    i0 = (s // 8) * IHALF
    pltpu.sync_copy(g2_sh.at[q], g2c_v)
    pltpu.make_async_copy(preds_hbm.at[pl.ds(i0, IHALF), :], ma_v, sem_ma).wait()

    ub = jnp.full((L,), -jnp.inf, jnp.float32)
    for row in range(IHALF):
        u = g2c_v[i0 + row, :]
        for off in WOFF:
            w = jnp.float32(off * off)
            dn = jnp.maximum(i0 + row - off, 0)
            up = jnp.minimum(i0 + row + off, N - 1)
            u = jnp.minimum(u, g2c_v[dn, :] + w)
            u = jnp.minimum(u, g2c_v[up, :] + w)
        m = ma_v[row, pl.ds(jc, L)]
        ub = jnp.maximum(ub, jnp.where(m > 0.5, u, -jnp.inf))

    stage_v[...] = ub
    pltpu.sync_copy(stage_v, u_sh.at[pl.ds(s * L, L)])
    plsc.subcore_barrier()
    pltpu.sync_copy(u_sh, u_v)

    bv = u_v[pl.ds(0, L)]
    for c in range(1, 16):
        bv = jnp.maximum(bv, u_v[pl.ds(c * L, L)])
    bound = jnp.max(bv)

    radius = jnp.int32(0)
    for c in range(N // L):
        dv = (lane + (c * L + 1)).astype(jnp.float32)
        radius = radius + jnp.sum(jnp.where(dv * dv <= bound, 1, 0))

    inf16 = jnp.full((L,), jnp.inf, jnp.float32)
    best = jnp.full((L,), -jnp.inf, jnp.float32)
    for ib in range(IHALF // IB):
        base_i = i0 + ib * IB
        lo = jnp.maximum(base_i - radius, 0)
        hi = jnp.minimum(base_i + (IB - 1) + radius + 1, N)

        def minplus(i2, accs):
            g = g2c_v[i2, :]
            out = []
            for k in range(IB):
                di = (i0 + ib * IB + k) - i2
                out.append(jnp.minimum(accs[k], g + (di * di).astype(jnp.float32)))
            return tuple(out)

        accs = lax.fori_loop(lo, hi, minplus, (inf16,) * IB)

        for k in range(IB):
            m = ma_v[ib * IB + k, pl.ds(jc, L)]
            best = jnp.maximum(best, jnp.where(m > 0.5, accs[k], -jnp.inf))

    stage_v[...] = best
    pltpu.sync_copy(stage_v, out_hbm.at[pl.ds((d * 16 + s) * L, L)])


def kernel(preds, targets):
    partials = _hausdorff_kernel(preds, targets)
    max_min = jnp.max(partials.reshape(2, 16 * L), axis=1)
    hd = jnp.sqrt(max_min)
    return jnp.maximum(hd[0], hd[1])
